# baseline (speedup 1.0000x reference)
import jax
import jax.numpy as jnp
from jax import lax
from jax.experimental import pallas as pl
from jax.experimental.pallas import tpu as pltpu
from jax.experimental.pallas import tpu_sc as plsc

EPS = 1e-05
LANES = 16
NUM_CORES = 2
NUM_SUBCORES = 16
NUM_WORKERS = NUM_CORES * NUM_SUBCORES
IDX_CHUNK = 128


def _sc_stage(theta, u3, v3, batch, dim, b_per_w):
    n_chunks = b_per_w // IDX_CHUNK
    n_groups = b_per_w // LANES
    mesh = plsc.VectorSubcoreMesh(core_axis_name="c", subcore_axis_name="s")

    def body(theta_hbm, u_hbm, v_hbm, d2_hbm, prod_hbm,
             idx_u, idx_v, eu, ev, d2_v, prod_v, sem):
        cid = lax.axis_index("c")
        sid = lax.axis_index("s")
        wid = sid * NUM_CORES + cid
        base = wid * b_per_w
        pltpu.sync_copy(u_hbm.at[wid], idx_u)
        pltpu.sync_copy(v_hbm.at[wid], idx_v)
        copies = []
        for j in range(n_chunks):
            copies.append(pltpu.async_copy(
                theta_hbm.at[idx_u.at[j]],
                eu.at[pl.ds(j * IDX_CHUNK, IDX_CHUNK)], sem))
            copies.append(pltpu.async_copy(
                theta_hbm.at[idx_v.at[j]],
                ev.at[pl.ds(j * IDX_CHUNK, IDX_CHUNK)], sem))
        for c in copies:
            c.wait()

        iota = lax.iota(jnp.int32, LANES)

        def group(g, carry):
            row = g * LANES + iota
            nu = jnp.zeros((LANES,), jnp.float32)
            nv = jnp.zeros((LANES,), jnp.float32)
            d2 = jnp.zeros((LANES,), jnp.float32)
            for d in range(dim):
                col = jnp.full((LANES,), d, jnp.int32)
                a = plsc.load_gather(eu, [row, col])
                b = plsc.load_gather(ev, [row, col])
                nu = nu + a * a
                nv = nv + b * b
                df = a - b
                d2 = d2 + df * df
            one_mu = 1.0 - jnp.minimum(nu, 1.0 - EPS)
            one_mv = 1.0 - jnp.minimum(nv, 1.0 - EPS)
            d2_v[pl.ds(g * LANES, LANES)] = d2
            prod_v[pl.ds(g * LANES, LANES)] = one_mu * one_mv
            return carry

        lax.fori_loop(0, n_groups, group, 0)
        pltpu.sync_copy(d2_v, d2_hbm.at[pl.ds(base, b_per_w)])
        pltpu.sync_copy(prod_v, prod_hbm.at[pl.ds(base, b_per_w)])

    f = pl.kernel(
        body,
        mesh=mesh,
        compiler_params=pltpu.CompilerParams(
            needs_layout_passes=False, use_tc_tiling_on_sc=False),
        out_type=(
            jax.ShapeDtypeStruct((batch,), jnp.float32),
            jax.ShapeDtypeStruct((batch,), jnp.float32),
        ),
        scratch_types=[
            pltpu.VMEM((n_chunks, IDX_CHUNK), jnp.int32),
            pltpu.VMEM((n_chunks, IDX_CHUNK), jnp.int32),
            pltpu.VMEM((b_per_w, dim), jnp.float32),
            pltpu.VMEM((b_per_w, dim), jnp.float32),
            pltpu.VMEM((b_per_w,), jnp.float32),
            pltpu.VMEM((b_per_w,), jnp.float32),
            pltpu.SemaphoreType.DMA,
        ],
    )
    return f(theta, u3, v3)


def _tc_body(r_ref, t_ref, d2_ref, prod_ref, o_ref):
    rr = r_ref[0]
    tt = t_ref[0]
    d2 = d2_ref[...]
    pr = prod_ref[...]
    s = 2.0 * jnp.sqrt(d2 + EPS) / pr
    duv = jnp.log(1.0 + s + jnp.sqrt(s * (s + 2.0)))
    o_ref[...] = 1.0 / (jnp.exp((duv - rr) / tt) + 1.0)


def kernel(u, v, theta, r, t):
    batch = u.shape[0]
    dim = theta.shape[1]
    b_per_w = batch // NUM_WORKERS
    u3 = u.reshape(NUM_WORKERS, b_per_w // IDX_CHUNK, IDX_CHUNK)
    v3 = v.reshape(NUM_WORKERS, b_per_w // IDX_CHUNK, IDX_CHUNK)
    d2, prod = _sc_stage(theta, u3, v3, batch, dim, b_per_w)

    rows = batch // 128
    out = pl.pallas_call(
        _tc_body,
        out_shape=jax.ShapeDtypeStruct((rows, 128), jnp.float32),
        in_specs=[
            pl.BlockSpec(memory_space=pltpu.SMEM),
            pl.BlockSpec(memory_space=pltpu.SMEM),
            pl.BlockSpec(memory_space=pltpu.VMEM),
            pl.BlockSpec(memory_space=pltpu.VMEM),
        ],
    )(r.reshape(1), t.reshape(1), d2.reshape(rows, 128), prod.reshape(rows, 128))
    return out.reshape(batch)


# speedup vs baseline: 1.0013x; 1.0013x over previous
import jax
import jax.numpy as jnp
from jax import lax
from jax.experimental import pallas as pl
from jax.experimental.pallas import tpu as pltpu
from jax.experimental.pallas import tpu_sc as plsc

EPS = 1e-05
LANES = 16
NUM_CORES = 2
NUM_SUBCORES = 16
NUM_WORKERS = NUM_CORES * NUM_SUBCORES
IDX_CHUNK = 128
BIG = 128


def _sc_stage(theta_big, u3, v3, batch, dim, b_per_w):
    n_chunks = b_per_w // IDX_CHUNK
    n_groups = IDX_CHUNK // LANES
    rows_per_big = BIG // dim
    mesh = plsc.VectorSubcoreMesh(core_axis_name="c", subcore_axis_name="s")

    def body(theta_hbm, u_hbm, v_hbm, d2_hbm, prod_hbm,
             idx_u, idx_v, big_u, big_v,
             buf_u0, buf_u1, buf_v0, buf_v1, d2_v, prod_v, sem0, sem1):
        cid = lax.axis_index("c")
        sid = lax.axis_index("s")
        wid = sid * NUM_CORES + cid
        base = wid * b_per_w
        pltpu.sync_copy(u_hbm.at[wid], idx_u)
        pltpu.sync_copy(v_hbm.at[wid], idx_v)

        def mk_big(s, carry):
            iu = idx_u[s // 8, pl.ds((s % 8) * LANES, LANES)]
            ivv = idx_v[s // 8, pl.ds((s % 8) * LANES, LANES)]
            big_u[s // 8, pl.ds((s % 8) * LANES, LANES)] = lax.shift_right_logical(iu, 2)
            big_v[s // 8, pl.ds((s % 8) * LANES, LANES)] = lax.shift_right_logical(ivv, 2)
            return carry

        lax.fori_loop(0, n_chunks * (IDX_CHUNK // LANES), mk_big, 0)

        bufs_u = (buf_u0, buf_u1)
        bufs_v = (buf_v0, buf_v1)
        sems = (sem0, sem1)
        iota = lax.iota(jnp.int32, LANES)

        def fire(j):
            s = sems[j % 2]
            return (
                pltpu.async_copy(theta_hbm.at[big_u.at[j]], bufs_u[j % 2], s),
                pltpu.async_copy(theta_hbm.at[big_v.at[j]], bufs_v[j % 2], s),
            )

        pending = {0: fire(0)}
        if n_chunks > 1:
            pending[1] = fire(1)

        for j in range(n_chunks):
            for c in pending.pop(j):
                c.wait()
            bu = bufs_u[j % 2]
            bv = bufs_v[j % 2]

            def group(g, carry, j=j, bu=bu, bv=bv):
                iu = idx_u[j, pl.ds(g * LANES, LANES)]
                ivv = idx_v[j, pl.ds(g * LANES, LANES)]
                cu = (iu & (rows_per_big - 1)) * dim
                cv = (ivv & (rows_per_big - 1)) * dim
                row = g * LANES + iota
                nu = jnp.zeros((LANES,), jnp.float32)
                nv = jnp.zeros((LANES,), jnp.float32)
                d2 = jnp.zeros((LANES,), jnp.float32)
                for d in range(dim):
                    a = plsc.load_gather(bu, [row, cu + d])
                    b = plsc.load_gather(bv, [row, cv + d])
                    nu = nu + a * a
                    nv = nv + b * b
                    df = a - b
                    d2 = d2 + df * df
                one_mu = 1.0 - jnp.minimum(nu, 1.0 - EPS)
                one_mv = 1.0 - jnp.minimum(nv, 1.0 - EPS)
                d2_v[pl.ds(j * IDX_CHUNK + g * LANES, LANES)] = d2
                prod_v[pl.ds(j * IDX_CHUNK + g * LANES, LANES)] = one_mu * one_mv
                return carry

            lax.fori_loop(0, n_groups, group, 0)
            if j + 2 < n_chunks:
                pending[j + 2] = fire(j + 2)

        pltpu.sync_copy(d2_v, d2_hbm.at[pl.ds(base, b_per_w)])
        pltpu.sync_copy(prod_v, prod_hbm.at[pl.ds(base, b_per_w)])

    f = pl.kernel(
        body,
        mesh=mesh,
        compiler_params=pltpu.CompilerParams(
            needs_layout_passes=False, use_tc_tiling_on_sc=True),
        out_type=(
            jax.ShapeDtypeStruct((batch,), jnp.float32),
            jax.ShapeDtypeStruct((batch,), jnp.float32),
        ),
        scratch_types=[
            pltpu.VMEM((n_chunks, IDX_CHUNK), jnp.int32),
            pltpu.VMEM((n_chunks, IDX_CHUNK), jnp.int32),
            pltpu.VMEM((n_chunks, IDX_CHUNK), jnp.int32),
            pltpu.VMEM((n_chunks, IDX_CHUNK), jnp.int32),
            pltpu.VMEM((IDX_CHUNK, BIG), jnp.float32),
            pltpu.VMEM((IDX_CHUNK, BIG), jnp.float32),
            pltpu.VMEM((IDX_CHUNK, BIG), jnp.float32),
            pltpu.VMEM((IDX_CHUNK, BIG), jnp.float32),
            pltpu.VMEM((b_per_w,), jnp.float32),
            pltpu.VMEM((b_per_w,), jnp.float32),
            pltpu.SemaphoreType.DMA,
            pltpu.SemaphoreType.DMA,
        ],
    )
    return f(theta_big, u3, v3)


def _tc_body(r_ref, t_ref, d2_ref, prod_ref, o_ref):
    rr = r_ref[0]
    tt = t_ref[0]
    d2 = d2_ref[...]
    pr = prod_ref[...]
    s = 2.0 * jnp.sqrt(d2 + EPS) / pr
    duv = jnp.log(1.0 + s + jnp.sqrt(s * (s + 2.0)))
    o_ref[...] = 1.0 / (jnp.exp((duv - rr) / tt) + 1.0)


def kernel(u, v, theta, r, t):
    batch = u.shape[0]
    dim = theta.shape[1]
    b_per_w = batch // NUM_WORKERS
    theta_big = theta.reshape(theta.shape[0] * dim // BIG, BIG)
    u3 = u.reshape(NUM_WORKERS, b_per_w // IDX_CHUNK, IDX_CHUNK)
    v3 = v.reshape(NUM_WORKERS, b_per_w // IDX_CHUNK, IDX_CHUNK)
    d2, prod = _sc_stage(theta_big, u3, v3, batch, dim, b_per_w)

    rows = batch // 128
    out = pl.pallas_call(
        _tc_body,
        out_shape=jax.ShapeDtypeStruct((rows, 128), jnp.float32),
        in_specs=[
            pl.BlockSpec(memory_space=pltpu.SMEM),
            pl.BlockSpec(memory_space=pltpu.SMEM),
            pl.BlockSpec(memory_space=pltpu.VMEM),
            pl.BlockSpec(memory_space=pltpu.VMEM),
        ],
    )(r.reshape(1), t.reshape(1), d2.reshape(rows, 128), prod.reshape(rows, 128))
    return out.reshape(batch)


# speedup vs baseline: 1.4050x; 1.4032x over previous
import jax
import jax.numpy as jnp
from jax import lax
from jax.experimental import pallas as pl
from jax.experimental.pallas import tpu as pltpu
from jax.experimental.pallas import tpu_sc as plsc

EPS = 1e-05
LANES = 16
NUM_CORES = 2
NUM_SUBCORES = 16
NUM_WORKERS = NUM_CORES * NUM_SUBCORES
BLK = 8
PASS_ROWS = 32


def _sc_stage(theta, u2, v2, batch, dim, b_per_w):
    n_pass = b_per_w // PASS_ROWS
    n_groups = PASS_ROWS // LANES
    mesh = plsc.VectorSubcoreMesh(core_axis_name="c", subcore_axis_name="s")

    def body(theta_hbm, u_hbm, v_hbm, d2_hbm, prod_hbm,
             iv_u, iv_v, stage_u, stage_v, d2_v, prod_v, sem):
        cid = lax.axis_index("c")
        sid = lax.axis_index("s")
        wid = sid * NUM_CORES + cid
        base = wid * b_per_w
        pltpu.sync_copy(u_hbm.at[wid], iv_u)
        pltpu.sync_copy(v_hbm.at[wid], iv_v)
        iota = lax.iota(jnp.int32, LANES)

        def do_pass(p, carry):
            poff = p * PASS_ROWS

            def fire(gg, c):
                iu = iv_u[pl.ds(poff + gg * LANES, LANES)]
                ivv = iv_v[pl.ds(poff + gg * LANES, LANES)]
                bu_v = lax.shift_right_logical(iu, 3) * BLK
                bv_v = lax.shift_right_logical(ivv, 3) * BLK
                for l in range(LANES):
                    bu = bu_v[l]
                    bv = bv_v[l]
                    dst = (gg * LANES + l) * BLK
                    pltpu.async_copy(
                        theta_hbm.at[pl.ds(pl.multiple_of(bu, BLK), BLK)],
                        stage_u.at[pl.ds(dst, BLK)], sem)
                    pltpu.async_copy(
                        theta_hbm.at[pl.ds(pl.multiple_of(bv, BLK), BLK)],
                        stage_v.at[pl.ds(dst, BLK)], sem)
                return c

            lax.fori_loop(0, PASS_ROWS // LANES, fire, 0)

            def drain(i, c):
                pltpu.make_async_copy(
                    theta_hbm.at[pl.ds(0, BLK)],
                    stage_u.at[pl.ds(0, BLK)], sem).wait()
                pltpu.make_async_copy(
                    theta_hbm.at[pl.ds(0, BLK)],
                    stage_v.at[pl.ds(0, BLK)], sem).wait()
                return c

            lax.fori_loop(0, PASS_ROWS, drain, 0)

            for g in range(n_groups):
                goff = poff + g * LANES
                iu = iv_u[pl.ds(goff, LANES)]
                ivv = iv_v[pl.ds(goff, LANES)]
                srow_u = (g * LANES + iota) * BLK + (iu & (BLK - 1))
                srow_v = (g * LANES + iota) * BLK + (ivv & (BLK - 1))
                nu = jnp.zeros((LANES,), jnp.float32)
                nv = jnp.zeros((LANES,), jnp.float32)
                d2 = jnp.zeros((LANES,), jnp.float32)
                for d in range(dim):
                    col = jnp.full((LANES,), d, jnp.int32)
                    a = plsc.load_gather(stage_u, [srow_u, col])
                    b = plsc.load_gather(stage_v, [srow_v, col])
                    nu = nu + a * a
                    nv = nv + b * b
                    df = a - b
                    d2 = d2 + df * df
                one_mu = 1.0 - jnp.minimum(nu, 1.0 - EPS)
                one_mv = 1.0 - jnp.minimum(nv, 1.0 - EPS)
                d2_v[pl.ds(goff, LANES)] = d2
                prod_v[pl.ds(goff, LANES)] = one_mu * one_mv
            return carry

        lax.fori_loop(0, n_pass, do_pass, 0)
        pltpu.sync_copy(d2_v, d2_hbm.at[pl.ds(base, b_per_w)])
        pltpu.sync_copy(prod_v, prod_hbm.at[pl.ds(base, b_per_w)])

    f = pl.kernel(
        body,
        mesh=mesh,
        compiler_params=pltpu.CompilerParams(
            needs_layout_passes=False, use_tc_tiling_on_sc=True),
        out_type=(
            jax.ShapeDtypeStruct((batch,), jnp.float32),
            jax.ShapeDtypeStruct((batch,), jnp.float32),
        ),
        scratch_types=[
            pltpu.VMEM((b_per_w,), jnp.int32),
            pltpu.VMEM((b_per_w,), jnp.int32),
            pltpu.VMEM((PASS_ROWS * BLK, dim), jnp.float32),
            pltpu.VMEM((PASS_ROWS * BLK, dim), jnp.float32),
            pltpu.VMEM((b_per_w,), jnp.float32),
            pltpu.VMEM((b_per_w,), jnp.float32),
            pltpu.SemaphoreType.DMA,
        ],
    )
    return f(theta, u2, v2)


def _tc_body(r_ref, t_ref, d2_ref, prod_ref, o_ref):
    rr = r_ref[0]
    tt = t_ref[0]
    d2 = d2_ref[...]
    pr = prod_ref[...]
    s = 2.0 * jnp.sqrt(d2 + EPS) / pr
    duv = jnp.log(1.0 + s + jnp.sqrt(s * (s + 2.0)))
    o_ref[...] = 1.0 / (jnp.exp((duv - rr) / tt) + 1.0)


def kernel(u, v, theta, r, t):
    batch = u.shape[0]
    dim = theta.shape[1]
    b_per_w = batch // NUM_WORKERS
    u2 = u.reshape(NUM_WORKERS, b_per_w)
    v2 = v.reshape(NUM_WORKERS, b_per_w)
    d2, prod = _sc_stage(theta, u2, v2, batch, dim, b_per_w)

    rows = batch // 128
    out = pl.pallas_call(
        _tc_body,
        out_shape=jax.ShapeDtypeStruct((rows, 128), jnp.float32),
        in_specs=[
            pl.BlockSpec(memory_space=pltpu.SMEM),
            pl.BlockSpec(memory_space=pltpu.SMEM),
            pl.BlockSpec(memory_space=pltpu.VMEM),
            pl.BlockSpec(memory_space=pltpu.VMEM),
        ],
    )(r.reshape(1), t.reshape(1), d2.reshape(rows, 128), prod.reshape(rows, 128))
    return out.reshape(batch)
